# Initial kernel scaffold; baseline (speedup 1.0000x reference)
import jax
import jax.numpy as jnp
from jax.experimental import pallas as pl


def kernel(rpn_cls_prob, rpn_bbox_pred, im_info, anchors):
    raise NotImplementedError("write your pallas kernel here")


# speedup vs baseline: 24.4991x; 24.4991x over previous
import functools

import jax
import jax.numpy as jnp
from jax import lax
from jax.experimental import pallas as pl
from jax.experimental.pallas import tpu as pltpu

_FEAT_STRIDE = 16.0
_PRE_NMS_TOP_N = 6000
_POST_NMS_TOP_N = 300
_NMS_THRESH = 0.7
_MIN_SIZE = 16.0
_LANES = 128
_NEG_FILL = -1e9


def _proposal_kernel(A, K, ROWS,
                     sc_ref, dx_ref, dy_ref, dw_ref, dh_ref,
                     w_ref, h_ref, cx_ref, cy_ref, im_ref,
                     out_ref,
                     x1_s, y1_s, x2_s, y2_s, ar_s, lv_s, key_s, lin_s):
    f32 = jnp.float32
    i32 = jnp.int32
    N = A * K

    im_h = im_ref[0, 0]
    im_w = im_ref[0, 1]
    im_scale = im_ref[0, 2]

    widths = w_ref[...]
    heights = h_ref[...]
    dx = dx_ref[...]
    dy = dy_ref[...]
    dw = jnp.clip(dw_ref[...], -4.0, 4.0)
    dh = jnp.clip(dh_ref[...], -4.0, 4.0)
    pcx = dx * widths + cx_ref[...]
    pcy = dy * heights + cy_ref[...]
    pw = jnp.exp(dw) * widths
    ph = jnp.exp(dh) * heights
    x1 = pcx - 0.5 * pw
    y1 = pcy - 0.5 * ph
    x2 = pcx + 0.5 * pw
    y2 = pcy + 0.5 * ph
    x1 = jnp.minimum(jnp.maximum(x1, 0.0), im_w - 1.0)
    y1 = jnp.minimum(jnp.maximum(y1, 0.0), im_h - 1.0)
    x2 = jnp.minimum(jnp.maximum(x2, 0.0), im_w - 1.0)
    y2 = jnp.minimum(jnp.maximum(y2, 0.0), im_h - 1.0)
    ws = x2 - x1 + 1.0
    hs = y2 - y1 + 1.0
    min_sz = _MIN_SIZE * im_scale
    s = jnp.where((ws >= min_sz) & (hs >= min_sz), sc_ref[...], f32(_NEG_FILL))

    x1_s[...] = x1
    y1_s[...] = y1
    x2_s[...] = x2
    y2_s[...] = y2
    ar_s[...] = ws * hs
    lv_s[...] = s

    ri = lax.broadcasted_iota(i32, (ROWS, _LANES), 0)
    ci = lax.broadcasted_iota(i32, (ROWS, _LANES), 1)
    fi = ri * _LANES + ci
    lin_s[...] = (fi % K) * A + fi // K

    bits = lax.bitcast_convert_type(s, i32)
    key_s[...] = jnp.where(s == f32(_NEG_FILL), i32(-1), bits)

    def vbody(_, lohi):
        lo, hi = lohi
        mid = lo + (hi - lo) // 2
        cnt = jnp.sum((key_s[...] > mid).astype(i32))
        big = cnt >= _PRE_NMS_TOP_N
        return (jnp.where(big, mid, lo), jnp.where(big, hi, mid))

    _, tk = lax.fori_loop(0, 34, vbody, (i32(-2), i32(1 << 30)))

    keyv = key_s[...]
    linv = lin_s[...]
    cnt_gt = jnp.sum((keyv > tk).astype(i32))
    deficit = _PRE_NMS_TOP_N - cnt_gt
    eq = keyv == tk

    def ibody(_, lohi):
        lo, hi = lohi
        mid = lo + (hi - lo) // 2
        cnt = jnp.sum((eq & (linv <= mid)).astype(i32))
        ge = cnt >= deficit
        return (jnp.where(ge, lo, mid), jnp.where(ge, mid, hi))

    _, tie_hi = lax.fori_loop(0, 17, ibody, (i32(-1), i32(N - 1)))

    topmask = (keyv > tk) | (eq & (linv <= tie_hi))
    lv_s[...] = jnp.where(topmask, lv_s[...], -jnp.inf)

    lv0 = lv_s[...]
    m0 = jnp.max(lv0)
    fp = jnp.min(jnp.where(lv0 == m0, linv, i32(1 << 30)))

    def nbody(i, carry):
        lv = lv_s[...]
        lin = lin_s[...]
        m = jnp.max(lv)
        cand = jnp.min(jnp.where(lv == m, lin, i32(1 << 30)))
        best = jnp.where(m == -jnp.inf, fp, cand)
        bm = lin == best
        bx1 = jnp.sum(jnp.where(bm, x1_s[...], 0.0))
        by1 = jnp.sum(jnp.where(bm, y1_s[...], 0.0))
        bx2 = jnp.sum(jnp.where(bm, x2_s[...], 0.0))
        by2 = jnp.sum(jnp.where(bm, y2_s[...], 0.0))
        bar = jnp.sum(jnp.where(bm, ar_s[...], 0.0))
        xx1 = jnp.maximum(bx1, x1_s[...])
        yy1 = jnp.maximum(by1, y1_s[...])
        xx2 = jnp.minimum(bx2, x2_s[...])
        yy2 = jnp.minimum(by2, y2_s[...])
        inter = jnp.maximum(xx2 - xx1 + 1.0, 0.0) * jnp.maximum(yy2 - yy1 + 1.0, 0.0)
        ovr = inter / (bar + ar_s[...] - inter)
        lv_s[...] = jnp.where((ovr > _NMS_THRESH) | bm, -jnp.inf, lv)
        lane = lax.broadcasted_iota(i32, (1, _LANES), 1)
        row = jnp.where(lane == 1, bx1,
              jnp.where(lane == 2, by1,
              jnp.where(lane == 3, bx2,
              jnp.where(lane == 4, by2, 0.0))))
        out_ref[pl.ds(i, 1), :] = row
        return carry

    lax.fori_loop(0, _POST_NMS_TOP_N, nbody, 0)


def kernel(rpn_cls_prob, rpn_bbox_pred, im_info, anchors):
    f32 = jnp.float32
    B, C2, H, W = rpn_cls_prob.shape
    A = C2 // 2
    K = H * W
    N = A * K
    ROWS = N // _LANES

    scores = rpn_cls_prob[0, A:, :, :].reshape(ROWS, _LANES)
    dl = rpn_bbox_pred[0].reshape(A, 4, K)
    dx = dl[:, 0, :].reshape(ROWS, _LANES)
    dy = dl[:, 1, :].reshape(ROWS, _LANES)
    dw = dl[:, 2, :].reshape(ROWS, _LANES)
    dh = dl[:, 3, :].reshape(ROWS, _LANES)

    anc = anchors.astype(f32)
    aw = anc[:, 2] - anc[:, 0] + 1.0
    ah = anc[:, 3] - anc[:, 1] + 1.0
    acx = anc[:, 0] + 0.5 * aw
    acy = anc[:, 1] + 0.5 * ah
    kk = jnp.arange(K, dtype=jnp.int32)
    sx = (kk % W).astype(f32) * _FEAT_STRIDE
    sy = (kk // W).astype(f32) * _FEAT_STRIDE
    widths = jnp.broadcast_to(aw[:, None], (A, K)).reshape(ROWS, _LANES)
    heights = jnp.broadcast_to(ah[:, None], (A, K)).reshape(ROWS, _LANES)
    ctrx = (acx[:, None] + sx[None, :]).reshape(ROWS, _LANES)
    ctry = (acy[:, None] + sy[None, :]).reshape(ROWS, _LANES)

    body = functools.partial(_proposal_kernel, A, K, ROWS)
    vspec = pl.BlockSpec(memory_space=pltpu.VMEM)
    out = pl.pallas_call(
        body,
        out_shape=jax.ShapeDtypeStruct((_POST_NMS_TOP_N, _LANES), f32),
        in_specs=[vspec] * 9 + [pl.BlockSpec(memory_space=pltpu.SMEM)],
        out_specs=vspec,
        scratch_shapes=[pltpu.VMEM((ROWS, _LANES), f32)] * 6
                       + [pltpu.VMEM((ROWS, _LANES), jnp.int32)] * 2,
    )(scores, dx, dy, dw, dh, widths, heights, ctrx, ctry,
      im_info.astype(f32))

    return out[None, :, :5]


# speedup vs baseline: 25.5643x; 1.0435x over previous
import functools

import jax
import jax.numpy as jnp
from jax import lax
from jax.experimental import pallas as pl
from jax.experimental.pallas import tpu as pltpu

_FEAT_STRIDE = 16.0
_PRE_NMS_TOP_N = 6000
_POST_NMS_TOP_N = 300
_NMS_THRESH = 0.7
_MIN_SIZE = 16.0
_LANES = 128
_NEG_FILL = -1e9


def _proposal_kernel(A, K, ROWS,
                     sc_ref, dx_ref, dy_ref, dw_ref, dh_ref,
                     w_ref, h_ref, cx_ref, cy_ref, im_ref,
                     out_ref,
                     x1_s, y1_s, x2_s, y2_s, ar_s, lv_s, key_s, lin_s):
    f32 = jnp.float32
    i32 = jnp.int32
    N = A * K

    im_h = im_ref[0, 0]
    im_w = im_ref[0, 1]
    im_scale = im_ref[0, 2]

    widths = w_ref[...]
    heights = h_ref[...]
    dx = dx_ref[...]
    dy = dy_ref[...]
    dw = jnp.clip(dw_ref[...], -4.0, 4.0)
    dh = jnp.clip(dh_ref[...], -4.0, 4.0)
    pcx = dx * widths + cx_ref[...]
    pcy = dy * heights + cy_ref[...]
    pw = jnp.exp(dw) * widths
    ph = jnp.exp(dh) * heights
    x1 = pcx - 0.5 * pw
    y1 = pcy - 0.5 * ph
    x2 = pcx + 0.5 * pw
    y2 = pcy + 0.5 * ph
    x1 = jnp.minimum(jnp.maximum(x1, 0.0), im_w - 1.0)
    y1 = jnp.minimum(jnp.maximum(y1, 0.0), im_h - 1.0)
    x2 = jnp.minimum(jnp.maximum(x2, 0.0), im_w - 1.0)
    y2 = jnp.minimum(jnp.maximum(y2, 0.0), im_h - 1.0)
    ws = x2 - x1 + 1.0
    hs = y2 - y1 + 1.0
    min_sz = _MIN_SIZE * im_scale
    s = jnp.where((ws >= min_sz) & (hs >= min_sz), sc_ref[...], f32(_NEG_FILL))

    x1_s[...] = x1
    y1_s[...] = y1
    x2_s[...] = x2
    y2_s[...] = y2
    ar_s[...] = ws * hs
    lv_s[...] = s

    ri = lax.broadcasted_iota(i32, (ROWS, _LANES), 0)
    ci = lax.broadcasted_iota(i32, (ROWS, _LANES), 1)
    fi = ri * _LANES + ci
    lin_s[...] = (fi % K) * A + fi // K

    bits = lax.bitcast_convert_type(s, i32)
    key_s[...] = jnp.where(s == f32(_NEG_FILL), i32(-1), bits)

    def vbody(_, lohi):
        lo, hi = lohi
        mid = lo + (hi - lo) // 2
        cnt = jnp.sum((key_s[...] > mid).astype(i32))
        big = cnt >= _PRE_NMS_TOP_N
        return (jnp.where(big, mid, lo), jnp.where(big, hi, mid))

    _, tk = lax.fori_loop(0, 34, vbody, (i32(-2), i32(1 << 30)))

    keyv = key_s[...]
    linv = lin_s[...]
    cnt_gt = jnp.sum((keyv > tk).astype(i32))
    deficit = _PRE_NMS_TOP_N - cnt_gt
    eq = keyv == tk

    def ibody(_, lohi):
        lo, hi = lohi
        mid = lo + (hi - lo) // 2
        cnt = jnp.sum((eq & (linv <= mid)).astype(i32))
        ge = cnt >= deficit
        return (jnp.where(ge, lo, mid), jnp.where(ge, mid, hi))

    _, tie_hi = lax.fori_loop(0, 17, ibody, (i32(-1), i32(N - 1)))

    topmask = (keyv > tk) | (eq & (linv <= tie_hi))
    lv_s[...] = jnp.where(topmask, lv_s[...], -jnp.inf)

    lv0 = lv_s[...]
    m0 = jnp.max(lv0)
    fp = jnp.min(jnp.where(lv0 == m0, linv, i32(1 << 30)))

    lane = lax.broadcasted_iota(i32, (1, _LANES), 1)

    def nbody(i, m):
        lv = lv_s[...]
        lin = lin_s[...]
        cand = jnp.min(jnp.where(lv == m, lin, i32(1 << 30)))
        best = jnp.where(m == -jnp.inf, fp, cand)
        f = (best % A) * K + best // A
        r = f // _LANES
        c = f % _LANES
        onehot = lane == c
        bx1 = jnp.sum(jnp.where(onehot, x1_s[pl.ds(r, 1), :], 0.0))
        by1 = jnp.sum(jnp.where(onehot, y1_s[pl.ds(r, 1), :], 0.0))
        bx2 = jnp.sum(jnp.where(onehot, x2_s[pl.ds(r, 1), :], 0.0))
        by2 = jnp.sum(jnp.where(onehot, y2_s[pl.ds(r, 1), :], 0.0))
        bar = jnp.sum(jnp.where(onehot, ar_s[pl.ds(r, 1), :], 0.0))
        xx1 = jnp.maximum(bx1, x1_s[...])
        yy1 = jnp.maximum(by1, y1_s[...])
        xx2 = jnp.minimum(bx2, x2_s[...])
        yy2 = jnp.minimum(by2, y2_s[...])
        inter = jnp.maximum(xx2 - xx1 + 1.0, 0.0) * jnp.maximum(yy2 - yy1 + 1.0, 0.0)
        ovr = inter / (bar + ar_s[...] - inter)
        new_lv = jnp.where((ovr > _NMS_THRESH) | (lin == best), -jnp.inf, lv)
        lv_s[...] = new_lv
        row = jnp.where(lane == 1, bx1,
              jnp.where(lane == 2, by1,
              jnp.where(lane == 3, bx2,
              jnp.where(lane == 4, by2, 0.0))))
        out_ref[pl.ds(i, 1), :] = row
        return jnp.max(new_lv)

    lax.fori_loop(0, _POST_NMS_TOP_N, nbody, m0)


def kernel(rpn_cls_prob, rpn_bbox_pred, im_info, anchors):
    f32 = jnp.float32
    B, C2, H, W = rpn_cls_prob.shape
    A = C2 // 2
    K = H * W
    N = A * K
    ROWS = N // _LANES

    scores = rpn_cls_prob[0, A:, :, :].reshape(ROWS, _LANES)
    dl = rpn_bbox_pred[0].reshape(A, 4, K)
    dx = dl[:, 0, :].reshape(ROWS, _LANES)
    dy = dl[:, 1, :].reshape(ROWS, _LANES)
    dw = dl[:, 2, :].reshape(ROWS, _LANES)
    dh = dl[:, 3, :].reshape(ROWS, _LANES)

    anc = anchors.astype(f32)
    aw = anc[:, 2] - anc[:, 0] + 1.0
    ah = anc[:, 3] - anc[:, 1] + 1.0
    acx = anc[:, 0] + 0.5 * aw
    acy = anc[:, 1] + 0.5 * ah
    kk = jnp.arange(K, dtype=jnp.int32)
    sx = (kk % W).astype(f32) * _FEAT_STRIDE
    sy = (kk // W).astype(f32) * _FEAT_STRIDE
    widths = jnp.broadcast_to(aw[:, None], (A, K)).reshape(ROWS, _LANES)
    heights = jnp.broadcast_to(ah[:, None], (A, K)).reshape(ROWS, _LANES)
    ctrx = (acx[:, None] + sx[None, :]).reshape(ROWS, _LANES)
    ctry = (acy[:, None] + sy[None, :]).reshape(ROWS, _LANES)

    body = functools.partial(_proposal_kernel, A, K, ROWS)
    vspec = pl.BlockSpec(memory_space=pltpu.VMEM)
    out = pl.pallas_call(
        body,
        out_shape=jax.ShapeDtypeStruct((_POST_NMS_TOP_N, _LANES), f32),
        in_specs=[vspec] * 9 + [pl.BlockSpec(memory_space=pltpu.SMEM)],
        out_specs=vspec,
        scratch_shapes=[pltpu.VMEM((ROWS, _LANES), f32)] * 6
                       + [pltpu.VMEM((ROWS, _LANES), jnp.int32)] * 2,
    )(scores, dx, dy, dw, dh, widths, heights, ctrx, ctry,
      im_info.astype(f32))

    return out[None, :, :5]


# speedup vs baseline: 35.1636x; 1.3755x over previous
import functools

import jax
import jax.numpy as jnp
from jax import lax
from jax.experimental import pallas as pl
from jax.experimental.pallas import tpu as pltpu

_FEAT_STRIDE = 16.0
_PRE_NMS_TOP_N = 6000
_POST_NMS_TOP_N = 300
_NMS_THRESH = 0.7
_MIN_SIZE = 16.0
_LANES = 128
_NEG_FILL = -1e9


def _proposal_kernel(A, K, ROWS,
                     sc_ref, dx_ref, dy_ref, dw_ref, dh_ref,
                     w_ref, h_ref, cx_ref, cy_ref, im_ref,
                     out_ref,
                     x1_s, y1_s, x2_s, y2_s, ar_s, lv_s, linf_s, key_s, lin_s):
    f32 = jnp.float32
    i32 = jnp.int32
    N = A * K

    im_h = im_ref[0, 0]
    im_w = im_ref[0, 1]
    im_scale = im_ref[0, 2]

    widths = w_ref[...]
    heights = h_ref[...]
    dx = dx_ref[...]
    dy = dy_ref[...]
    dw = jnp.clip(dw_ref[...], -4.0, 4.0)
    dh = jnp.clip(dh_ref[...], -4.0, 4.0)
    pcx = dx * widths + cx_ref[...]
    pcy = dy * heights + cy_ref[...]
    pw = jnp.exp(dw) * widths
    ph = jnp.exp(dh) * heights
    x1 = pcx - 0.5 * pw
    y1 = pcy - 0.5 * ph
    x2 = pcx + 0.5 * pw
    y2 = pcy + 0.5 * ph
    x1 = jnp.minimum(jnp.maximum(x1, 0.0), im_w - 1.0)
    y1 = jnp.minimum(jnp.maximum(y1, 0.0), im_h - 1.0)
    x2 = jnp.minimum(jnp.maximum(x2, 0.0), im_w - 1.0)
    y2 = jnp.minimum(jnp.maximum(y2, 0.0), im_h - 1.0)
    ws = x2 - x1 + 1.0
    hs = y2 - y1 + 1.0
    min_sz = _MIN_SIZE * im_scale
    s = jnp.where((ws >= min_sz) & (hs >= min_sz), sc_ref[...], f32(_NEG_FILL))

    x1_s[...] = x1
    y1_s[...] = y1
    x2_s[...] = x2
    y2_s[...] = y2
    ar_s[...] = ws * hs
    lv_s[...] = s

    ri = lax.broadcasted_iota(i32, (ROWS, _LANES), 0)
    ci = lax.broadcasted_iota(i32, (ROWS, _LANES), 1)
    fi = ri * _LANES + ci
    lin_s[...] = (fi % K) * A + fi // K
    linf_s[...] = lin_s[...].astype(f32)

    bits = lax.bitcast_convert_type(s, i32)
    key_s[...] = jnp.where(s == f32(_NEG_FILL), i32(-1), bits)

    def vbody(_, lohi):
        lo, hi = lohi
        mid = lo + (hi - lo) // 2
        cnt = jnp.sum((key_s[...] > mid).astype(i32))
        big = cnt >= _PRE_NMS_TOP_N
        return (jnp.where(big, mid, lo), jnp.where(big, hi, mid))

    _, tk = lax.fori_loop(0, 34, vbody, (i32(-2), i32(1 << 30)))

    keyv = key_s[...]
    linv = lin_s[...]
    cnt_gt = jnp.sum((keyv > tk).astype(i32))
    deficit = _PRE_NMS_TOP_N - cnt_gt
    eq = keyv == tk

    def ibody(_, lohi):
        lo, hi = lohi
        mid = lo + (hi - lo) // 2
        cnt = jnp.sum((eq & (linv <= mid)).astype(i32))
        ge = cnt >= deficit
        return (jnp.where(ge, lo, mid), jnp.where(ge, mid, hi))

    _, tie_hi = lax.fori_loop(0, 17, ibody, (i32(-1), i32(N - 1)))

    topmask = (keyv > tk) | (eq & (linv <= tie_hi))
    lv_s[...] = jnp.where(topmask, lv_s[...], -jnp.inf)

    lv0 = lv_s[...]
    m0 = jnp.max(lv0)
    fp = jnp.min(jnp.where(lv0 == m0, linv, i32(1 << 30)))

    lane = lax.broadcasted_iota(i32, (1, _LANES), 1)
    BIGF = f32(1 << 30)

    def nbody(i, m):
        lv = lv_s[...]
        lin = lin_s[...]
        bm0 = lv == m
        bm0f = jnp.where(bm0, 1.0, 0.0)
        cnt = jnp.sum(bm0f)
        negmin = jnp.max(jnp.where(bm0, -linf_s[...], -BIGF))
        sx1 = jnp.sum(bm0f * x1_s[...])
        sy1 = jnp.sum(bm0f * y1_s[...])
        sx2 = jnp.sum(bm0f * x2_s[...])
        sy2 = jnp.sum(bm0f * y2_s[...])
        sar = jnp.sum(bm0f * ar_s[...])
        cand = (-negmin).astype(i32)
        best = jnp.where(m == -jnp.inf, fp, cand)
        tie = (cnt != 1.0) | (m == -jnp.inf)

        def slow(_):
            f = (best % A) * K + best // A
            r = f // _LANES
            c = f % _LANES
            onehot = lane == c
            return (jnp.sum(jnp.where(onehot, x1_s[pl.ds(r, 1), :], 0.0)),
                    jnp.sum(jnp.where(onehot, y1_s[pl.ds(r, 1), :], 0.0)),
                    jnp.sum(jnp.where(onehot, x2_s[pl.ds(r, 1), :], 0.0)),
                    jnp.sum(jnp.where(onehot, y2_s[pl.ds(r, 1), :], 0.0)),
                    jnp.sum(jnp.where(onehot, ar_s[pl.ds(r, 1), :], 0.0)))

        def fast(_):
            return (sx1, sy1, sx2, sy2, sar)

        bx1, by1, bx2, by2, bar = lax.cond(tie, slow, fast, 0)
        xx1 = jnp.maximum(bx1, x1_s[...])
        yy1 = jnp.maximum(by1, y1_s[...])
        xx2 = jnp.minimum(bx2, x2_s[...])
        yy2 = jnp.minimum(by2, y2_s[...])
        inter = jnp.maximum(xx2 - xx1 + 1.0, 0.0) * jnp.maximum(yy2 - yy1 + 1.0, 0.0)
        ovr = inter / (bar + ar_s[...] - inter)
        new_lv = jnp.where((ovr > _NMS_THRESH) | (lin == best), -jnp.inf, lv)
        lv_s[...] = new_lv
        row = jnp.where(lane == 1, bx1,
              jnp.where(lane == 2, by1,
              jnp.where(lane == 3, bx2,
              jnp.where(lane == 4, by2, 0.0))))
        out_ref[pl.ds(i, 1), :] = row
        return jnp.max(new_lv)

    lax.fori_loop(0, _POST_NMS_TOP_N, nbody, m0)


def kernel(rpn_cls_prob, rpn_bbox_pred, im_info, anchors):
    f32 = jnp.float32
    B, C2, H, W = rpn_cls_prob.shape
    A = C2 // 2
    K = H * W
    N = A * K
    ROWS = N // _LANES

    scores = rpn_cls_prob[0, A:, :, :].reshape(ROWS, _LANES)
    dl = rpn_bbox_pred[0].reshape(A, 4, K)
    dx = dl[:, 0, :].reshape(ROWS, _LANES)
    dy = dl[:, 1, :].reshape(ROWS, _LANES)
    dw = dl[:, 2, :].reshape(ROWS, _LANES)
    dh = dl[:, 3, :].reshape(ROWS, _LANES)

    anc = anchors.astype(f32)
    aw = anc[:, 2] - anc[:, 0] + 1.0
    ah = anc[:, 3] - anc[:, 1] + 1.0
    acx = anc[:, 0] + 0.5 * aw
    acy = anc[:, 1] + 0.5 * ah
    kk = jnp.arange(K, dtype=jnp.int32)
    sx = (kk % W).astype(f32) * _FEAT_STRIDE
    sy = (kk // W).astype(f32) * _FEAT_STRIDE
    widths = jnp.broadcast_to(aw[:, None], (A, K)).reshape(ROWS, _LANES)
    heights = jnp.broadcast_to(ah[:, None], (A, K)).reshape(ROWS, _LANES)
    ctrx = (acx[:, None] + sx[None, :]).reshape(ROWS, _LANES)
    ctry = (acy[:, None] + sy[None, :]).reshape(ROWS, _LANES)

    body = functools.partial(_proposal_kernel, A, K, ROWS)
    vspec = pl.BlockSpec(memory_space=pltpu.VMEM)
    out = pl.pallas_call(
        body,
        out_shape=jax.ShapeDtypeStruct((_POST_NMS_TOP_N, _LANES), f32),
        in_specs=[vspec] * 9 + [pl.BlockSpec(memory_space=pltpu.SMEM)],
        out_specs=vspec,
        scratch_shapes=[pltpu.VMEM((ROWS, _LANES), f32)] * 7
                       + [pltpu.VMEM((ROWS, _LANES), jnp.int32)] * 2,
    )(scores, dx, dy, dw, dh, widths, heights, ctrx, ctry,
      im_info.astype(f32))

    return out[None, :, :5]


# speedup vs baseline: 38.5728x; 1.0970x over previous
import functools

import jax
import jax.numpy as jnp
from jax import lax
from jax.experimental import pallas as pl
from jax.experimental.pallas import tpu as pltpu

_FEAT_STRIDE = 16.0
_PRE_NMS_TOP_N = 6000
_POST_NMS_TOP_N = 300
_NMS_THRESH = 0.7
_MIN_SIZE = 16.0
_LANES = 128
_NEG_FILL = -1e9


def _proposal_kernel(A, K, ROWS,
                     sc_ref, dx_ref, dy_ref, dw_ref, dh_ref,
                     w_ref, h_ref, cx_ref, cy_ref, im_ref,
                     out_ref,
                     x1_s, y1_s, x2_s, y2_s, ar_s, lv_s, linf_s, key_s, lin_s):
    f32 = jnp.float32
    i32 = jnp.int32
    N = A * K

    im_h = im_ref[0, 0]
    im_w = im_ref[0, 1]
    im_scale = im_ref[0, 2]

    widths = w_ref[...]
    heights = h_ref[...]
    dx = dx_ref[...]
    dy = dy_ref[...]
    dw = jnp.clip(dw_ref[...], -4.0, 4.0)
    dh = jnp.clip(dh_ref[...], -4.0, 4.0)
    pcx = dx * widths + cx_ref[...]
    pcy = dy * heights + cy_ref[...]
    pw = jnp.exp(dw) * widths
    ph = jnp.exp(dh) * heights
    x1 = pcx - 0.5 * pw
    y1 = pcy - 0.5 * ph
    x2 = pcx + 0.5 * pw
    y2 = pcy + 0.5 * ph
    x1 = jnp.minimum(jnp.maximum(x1, 0.0), im_w - 1.0)
    y1 = jnp.minimum(jnp.maximum(y1, 0.0), im_h - 1.0)
    x2 = jnp.minimum(jnp.maximum(x2, 0.0), im_w - 1.0)
    y2 = jnp.minimum(jnp.maximum(y2, 0.0), im_h - 1.0)
    ws = x2 - x1 + 1.0
    hs = y2 - y1 + 1.0
    min_sz = _MIN_SIZE * im_scale
    s = jnp.where((ws >= min_sz) & (hs >= min_sz), sc_ref[...], f32(_NEG_FILL))

    x1_s[...] = x1
    y1_s[...] = y1
    x2_s[...] = x2
    y2_s[...] = y2
    ar_s[...] = ws * hs

    ri = lax.broadcasted_iota(i32, (ROWS, _LANES), 0)
    ci = lax.broadcasted_iota(i32, (ROWS, _LANES), 1)
    fi = ri * _LANES + ci
    lin_s[...] = (fi % K) * A + fi // K
    linf_s[...] = lin_s[...].astype(f32)

    bits = lax.bitcast_convert_type(s, i32)
    key_s[...] = jnp.where(s == f32(_NEG_FILL), i32(-1), bits)

    def vbody(_, lohi):
        lo, hi = lohi
        q = (hi - lo) // 4
        m1 = lo + q
        m2 = lo + 2 * q
        m3 = lo + 3 * q
        keyv = key_s[...]
        c1 = jnp.sum((keyv > m1).astype(i32), keepdims=True)
        c2 = jnp.sum((keyv > m2).astype(i32), keepdims=True)
        c3 = jnp.sum((keyv > m3).astype(i32), keepdims=True)
        P = _PRE_NMS_TOP_N
        nlo = jnp.where(c3 >= P, m3, jnp.where(c2 >= P, m2,
              jnp.where(c1 >= P, m1, lo)))
        nhi = jnp.where(c1 < P, m1, jnp.where(c2 < P, m2,
              jnp.where(c3 < P, m3, hi)))
        mid = lo + (hi - lo) // 2
        cmid = jnp.sum((key_s[...] > mid).astype(i32), keepdims=True)
        small = (hi - lo) < 4
        nlo = jnp.where(small, jnp.where(cmid >= P, mid, lo), nlo)
        nhi = jnp.where(small, jnp.where(cmid >= P, hi, mid), nhi)
        return (nlo, nhi)

    bis_lo = jnp.full((1, 1), -2, dtype=i32)
    bis_hi = jnp.full((1, 1), 1 << 30, dtype=i32)
    _, tk = lax.fori_loop(0, 18, vbody, (bis_lo, bis_hi))

    keyv = key_s[...]
    linv = lin_s[...]
    cnt_gt = jnp.sum((keyv > tk).astype(i32), keepdims=True)
    deficit = _PRE_NMS_TOP_N - cnt_gt
    eq = keyv == tk

    def ibody(_, lohi):
        lo, hi = lohi
        q = (hi - lo) // 4
        m1 = lo + q
        m2 = lo + 2 * q
        m3 = lo + 3 * q
        lv = lin_s[...]
        c1 = jnp.sum((eq & (lv <= m1)).astype(i32), keepdims=True)
        c2 = jnp.sum((eq & (lv <= m2)).astype(i32), keepdims=True)
        c3 = jnp.sum((eq & (lv <= m3)).astype(i32), keepdims=True)
        d = deficit
        nlo = jnp.where(c3 < d, m3, jnp.where(c2 < d, m2,
              jnp.where(c1 < d, m1, lo)))
        nhi = jnp.where(c1 >= d, m1, jnp.where(c2 >= d, m2,
              jnp.where(c3 >= d, m3, hi)))
        mid = lo + (hi - lo) // 2
        cmid = jnp.sum((eq & (lin_s[...] <= mid)).astype(i32), keepdims=True)
        small = (hi - lo) < 4
        nlo = jnp.where(small, jnp.where(cmid >= d, lo, mid), nlo)
        nhi = jnp.where(small, jnp.where(cmid >= d, mid, hi), nhi)
        return (nlo, nhi)

    ib_lo = jnp.full((1, 1), -1, dtype=i32)
    ib_hi = jnp.full((1, 1), N - 1, dtype=i32)
    _, tie_hi = lax.fori_loop(0, 10, ibody, (ib_lo, ib_hi))

    topmask = (keyv > tk) | (eq & (linv <= tie_hi))
    lv_s[...] = jnp.where(topmask, s, -jnp.inf)

    lv0 = lv_s[...]
    m0 = jnp.max(lv0, keepdims=True)
    fpf = jnp.max(jnp.where(lv0 == m0, -linf_s[...], -f32(1 << 30)), keepdims=True)

    lane = lax.broadcasted_iota(i32, (1, _LANES), 1)
    BIGF = f32(1 << 30)
    NINF = -jnp.inf

    def pair_body(j, m):
        lv = lv_s[...]
        linf = linf_s[...]
        x1 = x1_s[...]
        y1 = y1_s[...]
        x2 = x2_s[...]
        y2 = y2_s[...]
        ar = ar_s[...]
        bm1 = lv == m
        bm1f = jnp.where(bm1, 1.0, 0.0)
        cnt1 = jnp.sum(bm1f, keepdims=True)
        neg1 = jnp.max(jnp.where(bm1, -linf, -BIGF), keepdims=True)
        a_x1 = jnp.sum(bm1f * x1, keepdims=True)
        a_y1 = jnp.sum(bm1f * y1, keepdims=True)
        a_x2 = jnp.sum(bm1f * x2, keepdims=True)
        a_y2 = jnp.sum(bm1f * y2, keepdims=True)
        a_ar = jnp.sum(bm1f * ar, keepdims=True)
        m2 = jnp.max(jnp.where(bm1, NINF, lv), keepdims=True)
        nb1 = jnp.where(m == NINF, fpf, neg1)
        bm2 = (lv == m2) & (~bm1)
        bm2f = jnp.where(bm2, 1.0, 0.0)
        cnt2 = jnp.sum(bm2f, keepdims=True)
        nb2 = jnp.max(jnp.where(bm2, -linf, -BIGF), keepdims=True)
        b_x1 = jnp.sum(bm2f * x1, keepdims=True)
        b_y1 = jnp.sum(bm2f * y1, keepdims=True)
        b_x2 = jnp.sum(bm2f * x2, keepdims=True)
        b_y2 = jnp.sum(bm2f * y2, keepdims=True)
        b_ar = jnp.sum(bm2f * ar, keepdims=True)
        ix1 = jnp.maximum(a_x1, b_x1)
        iy1 = jnp.maximum(a_y1, b_y1)
        ix2 = jnp.minimum(a_x2, b_x2)
        iy2 = jnp.minimum(a_y2, b_y2)
        inter12 = jnp.maximum(ix2 - ix1 + 1.0, 0.0) * jnp.maximum(iy2 - iy1 + 1.0, 0.0)
        ovr12 = inter12 / (a_ar + b_ar - inter12)
        ok_v = ((cnt1 == 1.0) & (cnt2 == 1.0) & (m != NINF) & (m2 != NINF)
                & jnp.logical_not(ovr12 > _NMS_THRESH))
        okf = jnp.where(ok_v, 1.0, 0.0)
        ok = okf[0, 0] != 0.0

        def fast(_):
            xxa1 = jnp.maximum(a_x1, x1)
            yya1 = jnp.maximum(a_y1, y1)
            xxa2 = jnp.minimum(a_x2, x2)
            yya2 = jnp.minimum(a_y2, y2)
            ia = jnp.maximum(xxa2 - xxa1 + 1.0, 0.0) * jnp.maximum(yya2 - yya1 + 1.0, 0.0)
            ova = ia / (a_ar + ar - ia)
            xxb1 = jnp.maximum(b_x1, x1)
            yyb1 = jnp.maximum(b_y1, y1)
            xxb2 = jnp.minimum(b_x2, x2)
            yyb2 = jnp.minimum(b_y2, y2)
            ib = jnp.maximum(xxb2 - xxb1 + 1.0, 0.0) * jnp.maximum(yyb2 - yyb1 + 1.0, 0.0)
            ovb = ib / (b_ar + ar - ib)
            nl = jnp.where((ova > _NMS_THRESH) | (ovb > _NMS_THRESH)
                           | (-linf == nb1) | (-linf == nb2), NINF, lv)
            return (nl, a_x1, a_y1, a_x2, a_y2, b_x1, b_y1, b_x2, b_y2)

        def slow(_):
            fp = (-fpf[0, 0]).astype(i32)
            lin = lin_s[...]

            def one_pick(lvc, mc):
                candc = jnp.min(jnp.where(lvc == mc, lin, i32(1 << 30)))
                bestc = jnp.where(mc == NINF, fp, candc)
                fpos = (bestc % A) * K + bestc // A
                r = fpos // _LANES
                c = fpos % _LANES
                oh = lane == c
                cx1 = jnp.sum(jnp.where(oh, x1_s[pl.ds(r, 1), :], 0.0), keepdims=True)
                cy1 = jnp.sum(jnp.where(oh, y1_s[pl.ds(r, 1), :], 0.0), keepdims=True)
                cx2 = jnp.sum(jnp.where(oh, x2_s[pl.ds(r, 1), :], 0.0), keepdims=True)
                cy2 = jnp.sum(jnp.where(oh, y2_s[pl.ds(r, 1), :], 0.0), keepdims=True)
                car = jnp.sum(jnp.where(oh, ar_s[pl.ds(r, 1), :], 0.0), keepdims=True)
                u1 = jnp.maximum(cx1, x1)
                v1 = jnp.maximum(cy1, y1)
                u2 = jnp.minimum(cx2, x2)
                v2 = jnp.minimum(cy2, y2)
                it = jnp.maximum(u2 - u1 + 1.0, 0.0) * jnp.maximum(v2 - v1 + 1.0, 0.0)
                ov = it / (car + ar - it)
                nl = jnp.where((ov > _NMS_THRESH) | (lin == bestc), NINF, lvc)
                return nl, (cx1, cy1, cx2, cy2)

            nl1, boxa = one_pick(lv, m[0, 0])
            m1b = jnp.max(nl1)
            nl2, boxb = one_pick(nl1, m1b)
            return (nl2,) + boxa + boxb

        res = lax.cond(ok, fast, slow, 0)
        new_lv = res[0]
        r1x1, r1y1, r1x2, r1y2, r2x1, r2y1, r2x2, r2y2 = res[1:]
        lv_s[...] = new_lv
        row1 = jnp.where(lane == 1, r1x1, jnp.where(lane == 2, r1y1,
               jnp.where(lane == 3, r1x2, jnp.where(lane == 4, r1y2, 0.0))))
        row2 = jnp.where(lane == 1, r2x1, jnp.where(lane == 2, r2y1,
               jnp.where(lane == 3, r2x2, jnp.where(lane == 4, r2y2, 0.0))))
        out_ref[pl.ds(2 * j, 1), :] = row1
        out_ref[pl.ds(2 * j + 1, 1), :] = row2
        return jnp.max(new_lv, keepdims=True)

    lax.fori_loop(0, _POST_NMS_TOP_N // 2, pair_body, m0)


def kernel(rpn_cls_prob, rpn_bbox_pred, im_info, anchors):
    f32 = jnp.float32
    B, C2, H, W = rpn_cls_prob.shape
    A = C2 // 2
    K = H * W
    N = A * K
    ROWS = N // _LANES

    scores = rpn_cls_prob[0, A:, :, :].reshape(ROWS, _LANES)
    dl = rpn_bbox_pred[0].reshape(A, 4, K)
    dx = dl[:, 0, :].reshape(ROWS, _LANES)
    dy = dl[:, 1, :].reshape(ROWS, _LANES)
    dw = dl[:, 2, :].reshape(ROWS, _LANES)
    dh = dl[:, 3, :].reshape(ROWS, _LANES)

    anc = anchors.astype(f32)
    aw = anc[:, 2] - anc[:, 0] + 1.0
    ah = anc[:, 3] - anc[:, 1] + 1.0
    acx = anc[:, 0] + 0.5 * aw
    acy = anc[:, 1] + 0.5 * ah
    kk = jnp.arange(K, dtype=jnp.int32)
    sx = (kk % W).astype(f32) * _FEAT_STRIDE
    sy = (kk // W).astype(f32) * _FEAT_STRIDE
    widths = jnp.broadcast_to(aw[:, None], (A, K)).reshape(ROWS, _LANES)
    heights = jnp.broadcast_to(ah[:, None], (A, K)).reshape(ROWS, _LANES)
    ctrx = (acx[:, None] + sx[None, :]).reshape(ROWS, _LANES)
    ctry = (acy[:, None] + sy[None, :]).reshape(ROWS, _LANES)

    body = functools.partial(_proposal_kernel, A, K, ROWS)
    vspec = pl.BlockSpec(memory_space=pltpu.VMEM)
    out = pl.pallas_call(
        body,
        out_shape=jax.ShapeDtypeStruct((_POST_NMS_TOP_N, _LANES), f32),
        in_specs=[vspec] * 9 + [pl.BlockSpec(memory_space=pltpu.SMEM)],
        out_specs=vspec,
        scratch_shapes=[pltpu.VMEM((ROWS, _LANES), f32)] * 7
                       + [pltpu.VMEM((ROWS, _LANES), jnp.int32)] * 2,
    )(scores, dx, dy, dw, dh, widths, heights, ctrx, ctry,
      im_info.astype(f32))

    return out[None, :, :5]


# speedup vs baseline: 40.5656x; 1.0517x over previous
import functools

import jax
import jax.numpy as jnp
from jax import lax
from jax.experimental import pallas as pl
from jax.experimental.pallas import tpu as pltpu

_FEAT_STRIDE = 16.0
_PRE_NMS_TOP_N = 6000
_POST_NMS_TOP_N = 300
_NMS_THRESH = 0.7
_MIN_SIZE = 16.0
_LANES = 128
_NEG_FILL = -1e9


def _proposal_kernel(A, K, ROWS,
                     sc_ref, dx_ref, dy_ref, dw_ref, dh_ref,
                     w_ref, h_ref, cx_ref, cy_ref, im_ref,
                     out_ref,
                     x1_s, y1_s, x2_s, y2_s, ar_s, lv_s, linf_s, key_s, lin_s):
    f32 = jnp.float32
    i32 = jnp.int32
    N = A * K

    im_h = im_ref[0, 0]
    im_w = im_ref[0, 1]
    im_scale = im_ref[0, 2]

    widths = w_ref[...]
    heights = h_ref[...]
    dx = dx_ref[...]
    dy = dy_ref[...]
    dw = jnp.clip(dw_ref[...], -4.0, 4.0)
    dh = jnp.clip(dh_ref[...], -4.0, 4.0)
    pcx = dx * widths + cx_ref[...]
    pcy = dy * heights + cy_ref[...]
    pw = jnp.exp(dw) * widths
    ph = jnp.exp(dh) * heights
    x1 = pcx - 0.5 * pw
    y1 = pcy - 0.5 * ph
    x2 = pcx + 0.5 * pw
    y2 = pcy + 0.5 * ph
    x1 = jnp.minimum(jnp.maximum(x1, 0.0), im_w - 1.0)
    y1 = jnp.minimum(jnp.maximum(y1, 0.0), im_h - 1.0)
    x2 = jnp.minimum(jnp.maximum(x2, 0.0), im_w - 1.0)
    y2 = jnp.minimum(jnp.maximum(y2, 0.0), im_h - 1.0)
    ws = x2 - x1 + 1.0
    hs = y2 - y1 + 1.0
    min_sz = _MIN_SIZE * im_scale
    s = jnp.where((ws >= min_sz) & (hs >= min_sz), sc_ref[...], f32(_NEG_FILL))

    x1_s[...] = x1
    y1_s[...] = y1
    x2_s[...] = x2
    y2_s[...] = y2
    ar_s[...] = ws * hs

    ri = lax.broadcasted_iota(i32, (ROWS, _LANES), 0)
    ci = lax.broadcasted_iota(i32, (ROWS, _LANES), 1)
    fi = ri * _LANES + ci
    lin_s[...] = (fi % K) * A + fi // K
    linf_s[...] = lin_s[...].astype(f32)

    bits = lax.bitcast_convert_type(s, i32)
    key_s[...] = jnp.where(s == f32(_NEG_FILL), i32(-1), bits)

    def vbody(_, lohi):
        lo, hi = lohi
        q = (hi - lo) // 4
        m1 = lo + q
        m2 = lo + 2 * q
        m3 = lo + 3 * q
        keyv = key_s[...]
        c1 = jnp.sum((keyv > m1).astype(i32), keepdims=True)
        c2 = jnp.sum((keyv > m2).astype(i32), keepdims=True)
        c3 = jnp.sum((keyv > m3).astype(i32), keepdims=True)
        P = _PRE_NMS_TOP_N
        nlo = jnp.where(c3 >= P, m3, jnp.where(c2 >= P, m2,
              jnp.where(c1 >= P, m1, lo)))
        nhi = jnp.where(c1 < P, m1, jnp.where(c2 < P, m2,
              jnp.where(c3 < P, m3, hi)))
        mid = lo + (hi - lo) // 2
        cmid = jnp.sum((key_s[...] > mid).astype(i32), keepdims=True)
        small = (hi - lo) < 4
        nlo = jnp.where(small, jnp.where(cmid >= P, mid, lo), nlo)
        nhi = jnp.where(small, jnp.where(cmid >= P, hi, mid), nhi)
        return (nlo, nhi)

    bis_lo = jnp.full((1, 1), -2, dtype=i32)
    bis_hi = jnp.full((1, 1), 1 << 30, dtype=i32)
    _, tk = lax.fori_loop(0, 18, vbody, (bis_lo, bis_hi))

    keyv = key_s[...]
    linv = lin_s[...]
    cnt_gt = jnp.sum((keyv > tk).astype(i32), keepdims=True)
    deficit = _PRE_NMS_TOP_N - cnt_gt
    eq = keyv == tk

    def ibody(_, lohi):
        lo, hi = lohi
        q = (hi - lo) // 4
        m1 = lo + q
        m2 = lo + 2 * q
        m3 = lo + 3 * q
        lv = lin_s[...]
        c1 = jnp.sum((eq & (lv <= m1)).astype(i32), keepdims=True)
        c2 = jnp.sum((eq & (lv <= m2)).astype(i32), keepdims=True)
        c3 = jnp.sum((eq & (lv <= m3)).astype(i32), keepdims=True)
        d = deficit
        nlo = jnp.where(c3 < d, m3, jnp.where(c2 < d, m2,
              jnp.where(c1 < d, m1, lo)))
        nhi = jnp.where(c1 >= d, m1, jnp.where(c2 >= d, m2,
              jnp.where(c3 >= d, m3, hi)))
        mid = lo + (hi - lo) // 2
        cmid = jnp.sum((eq & (lin_s[...] <= mid)).astype(i32), keepdims=True)
        small = (hi - lo) < 4
        nlo = jnp.where(small, jnp.where(cmid >= d, lo, mid), nlo)
        nhi = jnp.where(small, jnp.where(cmid >= d, mid, hi), nhi)
        return (nlo, nhi)

    ib_lo = jnp.full((1, 1), -1, dtype=i32)
    ib_hi = jnp.full((1, 1), N - 1, dtype=i32)
    _, tie_hi = lax.fori_loop(0, 10, ibody, (ib_lo, ib_hi))

    topmask = (keyv > tk) | (eq & (linv <= tie_hi))
    lv_s[...] = jnp.where(topmask, s, -jnp.inf)

    lv0 = lv_s[...]
    m0 = jnp.max(lv0, keepdims=True)
    fpf = jnp.max(jnp.where(lv0 == m0, -linf_s[...], -f32(1 << 30)), keepdims=True)

    lane = lax.broadcasted_iota(i32, (1, _LANES), 1)
    BIGF = f32(1 << 30)
    NINF = -jnp.inf

    def pair_body(j, m):
        lv = lv_s[...]
        linf = linf_s[...]
        x1 = x1_s[...]
        y1 = y1_s[...]
        x2 = x2_s[...]
        y2 = y2_s[...]
        ar = ar_s[...]
        bm1 = lv == m
        bm1f = jnp.where(bm1, 1.0, 0.0)
        cnt1 = jnp.sum(bm1f, keepdims=True)
        neg1 = jnp.max(jnp.where(bm1, -linf, -BIGF), keepdims=True)
        a_x1 = jnp.sum(bm1f * x1, keepdims=True)
        a_y1 = jnp.sum(bm1f * y1, keepdims=True)
        a_x2 = jnp.sum(bm1f * x2, keepdims=True)
        a_y2 = jnp.sum(bm1f * y2, keepdims=True)
        a_ar = jnp.sum(bm1f * ar, keepdims=True)
        m2 = jnp.max(jnp.where(bm1, NINF, lv), keepdims=True)
        nb1 = jnp.where(m == NINF, fpf, neg1)
        bm2 = (lv == m2) & (~bm1)
        bm2f = jnp.where(bm2, 1.0, 0.0)
        cnt2 = jnp.sum(bm2f, keepdims=True)
        nb2 = jnp.max(jnp.where(bm2, -linf, -BIGF), keepdims=True)
        b_x1 = jnp.sum(bm2f * x1, keepdims=True)
        b_y1 = jnp.sum(bm2f * y1, keepdims=True)
        b_x2 = jnp.sum(bm2f * x2, keepdims=True)
        b_y2 = jnp.sum(bm2f * y2, keepdims=True)
        b_ar = jnp.sum(bm2f * ar, keepdims=True)
        bm12 = bm1 | bm2
        m3 = jnp.max(jnp.where(bm12, NINF, lv), keepdims=True)
        bm3 = (lv == m3) & (~bm12)
        bm3f = jnp.where(bm3, 1.0, 0.0)
        cnt3 = jnp.sum(bm3f, keepdims=True)
        nb3 = jnp.max(jnp.where(bm3, -linf, -BIGF), keepdims=True)
        c_x1 = jnp.sum(bm3f * x1, keepdims=True)
        c_y1 = jnp.sum(bm3f * y1, keepdims=True)
        c_x2 = jnp.sum(bm3f * x2, keepdims=True)
        c_y2 = jnp.sum(bm3f * y2, keepdims=True)
        c_ar = jnp.sum(bm3f * ar, keepdims=True)

        def iou11(px1, py1, px2, py2, par, qx1, qy1, qx2, qy2, qar):
            u1 = jnp.maximum(px1, qx1)
            v1 = jnp.maximum(py1, qy1)
            u2 = jnp.minimum(px2, qx2)
            v2 = jnp.minimum(py2, qy2)
            it = jnp.maximum(u2 - u1 + 1.0, 0.0) * jnp.maximum(v2 - v1 + 1.0, 0.0)
            return it / (par + qar - it)

        ovr12 = iou11(a_x1, a_y1, a_x2, a_y2, a_ar, b_x1, b_y1, b_x2, b_y2, b_ar)
        ovr13 = iou11(a_x1, a_y1, a_x2, a_y2, a_ar, c_x1, c_y1, c_x2, c_y2, c_ar)
        ovr23 = iou11(b_x1, b_y1, b_x2, b_y2, b_ar, c_x1, c_y1, c_x2, c_y2, c_ar)
        ok_v = ((cnt1 == 1.0) & (cnt2 == 1.0) & (cnt3 == 1.0)
                & (m != NINF) & (m2 != NINF) & (m3 != NINF)
                & jnp.logical_not(ovr12 > _NMS_THRESH)
                & jnp.logical_not(ovr13 > _NMS_THRESH)
                & jnp.logical_not(ovr23 > _NMS_THRESH))
        okf = jnp.where(ok_v, 1.0, 0.0)
        ok = okf[0, 0] != 0.0

        def sweep1(px1, py1, px2, py2, par):
            u1 = jnp.maximum(px1, x1)
            v1 = jnp.maximum(py1, y1)
            u2 = jnp.minimum(px2, x2)
            v2 = jnp.minimum(py2, y2)
            it = jnp.maximum(u2 - u1 + 1.0, 0.0) * jnp.maximum(v2 - v1 + 1.0, 0.0)
            return it / (par + ar - it)

        def fast(_):
            ova = sweep1(a_x1, a_y1, a_x2, a_y2, a_ar)
            ovb = sweep1(b_x1, b_y1, b_x2, b_y2, b_ar)
            ovc = sweep1(c_x1, c_y1, c_x2, c_y2, c_ar)
            nl = jnp.where((ova > _NMS_THRESH) | (ovb > _NMS_THRESH)
                           | (ovc > _NMS_THRESH) | (-linf == nb1)
                           | (-linf == nb2) | (-linf == nb3), NINF, lv)
            return (nl, a_x1, a_y1, a_x2, a_y2,
                    b_x1, b_y1, b_x2, b_y2, c_x1, c_y1, c_x2, c_y2)

        def slow(_):
            fp = (-fpf[0, 0]).astype(i32)
            lin = lin_s[...]

            def one_pick(lvc, mc):
                candc = jnp.min(jnp.where(lvc == mc, lin, i32(1 << 30)))
                bestc = jnp.where(mc == NINF, fp, candc)
                fpos = (bestc % A) * K + bestc // A
                r = fpos // _LANES
                c = fpos % _LANES
                oh = lane == c
                cx1 = jnp.sum(jnp.where(oh, x1_s[pl.ds(r, 1), :], 0.0), keepdims=True)
                cy1 = jnp.sum(jnp.where(oh, y1_s[pl.ds(r, 1), :], 0.0), keepdims=True)
                cx2 = jnp.sum(jnp.where(oh, x2_s[pl.ds(r, 1), :], 0.0), keepdims=True)
                cy2 = jnp.sum(jnp.where(oh, y2_s[pl.ds(r, 1), :], 0.0), keepdims=True)
                car = jnp.sum(jnp.where(oh, ar_s[pl.ds(r, 1), :], 0.0), keepdims=True)
                u1 = jnp.maximum(cx1, x1)
                v1 = jnp.maximum(cy1, y1)
                u2 = jnp.minimum(cx2, x2)
                v2 = jnp.minimum(cy2, y2)
                it = jnp.maximum(u2 - u1 + 1.0, 0.0) * jnp.maximum(v2 - v1 + 1.0, 0.0)
                ov = it / (car + ar - it)
                nl = jnp.where((ov > _NMS_THRESH) | (lin == bestc), NINF, lvc)
                return nl, (cx1, cy1, cx2, cy2)

            nl1, boxa = one_pick(lv, m[0, 0])
            nl2, boxb = one_pick(nl1, jnp.max(nl1))
            nl3, boxc = one_pick(nl2, jnp.max(nl2))
            return (nl3,) + boxa + boxb + boxc

        res = lax.cond(ok, fast, slow, 0)
        new_lv = res[0]
        (r1x1, r1y1, r1x2, r1y2, r2x1, r2y1, r2x2, r2y2,
         r3x1, r3y1, r3x2, r3y2) = res[1:]
        lv_s[...] = new_lv
        row1 = jnp.where(lane == 1, r1x1, jnp.where(lane == 2, r1y1,
               jnp.where(lane == 3, r1x2, jnp.where(lane == 4, r1y2, 0.0))))
        row2 = jnp.where(lane == 1, r2x1, jnp.where(lane == 2, r2y1,
               jnp.where(lane == 3, r2x2, jnp.where(lane == 4, r2y2, 0.0))))
        row3 = jnp.where(lane == 1, r3x1, jnp.where(lane == 2, r3y1,
               jnp.where(lane == 3, r3x2, jnp.where(lane == 4, r3y2, 0.0))))
        out_ref[pl.ds(3 * j, 1), :] = row1
        out_ref[pl.ds(3 * j + 1, 1), :] = row2
        out_ref[pl.ds(3 * j + 2, 1), :] = row3
        return jnp.max(new_lv, keepdims=True)

    lax.fori_loop(0, _POST_NMS_TOP_N // 3, pair_body, m0)


def kernel(rpn_cls_prob, rpn_bbox_pred, im_info, anchors):
    f32 = jnp.float32
    B, C2, H, W = rpn_cls_prob.shape
    A = C2 // 2
    K = H * W
    N = A * K
    ROWS = N // _LANES

    scores = rpn_cls_prob[0, A:, :, :].reshape(ROWS, _LANES)
    dl = rpn_bbox_pred[0].reshape(A, 4, K)
    dx = dl[:, 0, :].reshape(ROWS, _LANES)
    dy = dl[:, 1, :].reshape(ROWS, _LANES)
    dw = dl[:, 2, :].reshape(ROWS, _LANES)
    dh = dl[:, 3, :].reshape(ROWS, _LANES)

    anc = anchors.astype(f32)
    aw = anc[:, 2] - anc[:, 0] + 1.0
    ah = anc[:, 3] - anc[:, 1] + 1.0
    acx = anc[:, 0] + 0.5 * aw
    acy = anc[:, 1] + 0.5 * ah
    kk = jnp.arange(K, dtype=jnp.int32)
    sx = (kk % W).astype(f32) * _FEAT_STRIDE
    sy = (kk // W).astype(f32) * _FEAT_STRIDE
    widths = jnp.broadcast_to(aw[:, None], (A, K)).reshape(ROWS, _LANES)
    heights = jnp.broadcast_to(ah[:, None], (A, K)).reshape(ROWS, _LANES)
    ctrx = (acx[:, None] + sx[None, :]).reshape(ROWS, _LANES)
    ctry = (acy[:, None] + sy[None, :]).reshape(ROWS, _LANES)

    body = functools.partial(_proposal_kernel, A, K, ROWS)
    vspec = pl.BlockSpec(memory_space=pltpu.VMEM)
    out = pl.pallas_call(
        body,
        out_shape=jax.ShapeDtypeStruct((_POST_NMS_TOP_N, _LANES), f32),
        in_specs=[vspec] * 9 + [pl.BlockSpec(memory_space=pltpu.SMEM)],
        out_specs=vspec,
        scratch_shapes=[pltpu.VMEM((ROWS, _LANES), f32)] * 7
                       + [pltpu.VMEM((ROWS, _LANES), jnp.int32)] * 2,
    )(scores, dx, dy, dw, dh, widths, heights, ctrx, ctry,
      im_info.astype(f32))

    return out[None, :, :5]
